# Initial kernel scaffold; baseline (speedup 1.0000x reference)
import jax
import jax.numpy as jnp
from jax.experimental import pallas as pl


def kernel(user_id, item_id, category_id, shop_id, hist_item_id, target_item_id, w_user_id, w_item_id, w_category_id, w_shop_id, w_hist_item_id, w_target_item_id, bias):
    raise NotImplementedError("write your pallas kernel here")


# speedup vs baseline: 58.8418x; 58.8418x over previous
import functools

import jax
import jax.numpy as jnp
from jax import lax
from jax.experimental import pallas as pl
from jax.experimental.pallas import tpu as pltpu
from jax.experimental.pallas import tpu_sc as plsc

B = 16384
L = 20
NBUCKETS = 100000
NFEAT = 6

NC = 2
NS = 16
NW = NC * NS

CHUNK = 256
CPF = B // CHUNK
NCHUNKS = NFEAT * CPF
CPW = NCHUNKS // NW
GPC = CHUNK // 16


def _hash16(x):
    h = x.astype(jnp.uint32)
    h = h * jnp.uint32(2654435761)
    h = h ^ (h >> 16)
    h = h * jnp.uint32(2246822519)
    h = h ^ (h >> 13)
    return (h % jnp.uint32(NBUCKETS)).astype(jnp.int32)


def _sc_body(ids_hbm, w_hbm, part_hbm, table_v, ids_v, out_v):
    wid = lax.axis_index("c") * NS + lax.axis_index("s")
    c_lo = wid * CPW
    c_hi = c_lo + CPW

    def do_chunk(c, _):
        f = c // CPF
        r0 = (c % CPF) * CHUNK
        pltpu.sync_copy(ids_hbm.at[f, pl.ds(r0 * L, CHUNK * L)], ids_v)

        def group(g, _):
            rows = g * 16 + lax.iota(jnp.int32, 16)
            base = rows * L
            hs = []
            acc = jnp.zeros((16,), jnp.float32)
            for j in range(L):
                idj = plsc.load_gather(ids_v, [base + j])
                h = _hash16(idj)
                wj = plsc.load_gather(table_v, [h])
                if j == 0:
                    acc = wj
                else:
                    m = hs[0] != h
                    for k in range(1, j):
                        m = m & (hs[k] != h)
                    acc = acc + jnp.where(m, wj, 0.0)
                hs.append(h)
            out_v[pl.ds(g * 16, 16)] = acc
            return 0

        lax.fori_loop(0, GPC, group, 0)
        pltpu.sync_copy(out_v, part_hbm.at[f, pl.ds(r0, CHUNK)])
        return 0

    f0 = c_lo // CPF
    f1 = (c_hi - 1) // CPF
    split = jnp.minimum(c_hi, (f0 + 1) * CPF)

    pltpu.sync_copy(w_hbm.at[f0], table_v)
    lax.fori_loop(c_lo, split, do_chunk, 0)

    @pl.when(f1 != f0)
    def _second_feature():
        pltpu.sync_copy(w_hbm.at[f1], table_v)
        lax.fori_loop(split, c_hi, do_chunk, 0)


@jax.jit
def _sc_partials(ids_all, w_all):
    mesh = plsc.VectorSubcoreMesh(core_axis_name="c", subcore_axis_name="s")
    return pl.kernel(
        _sc_body,
        out_type=jax.ShapeDtypeStruct((NFEAT, B), jnp.float32),
        mesh=mesh,
        scratch_types=[
            pltpu.VMEM((NBUCKETS,), jnp.float32),
            pltpu.VMEM((CHUNK * L,), jnp.int32),
            pltpu.VMEM((CHUNK,), jnp.float32),
        ],
        compiler_params=pltpu.CompilerParams(needs_layout_passes=False),
    )(ids_all, w_all)


def _epilogue_body(part_ref, bias_ref, out_ref):
    out_ref[:, :] = jnp.sum(part_ref[:, :], axis=0, keepdims=True) + bias_ref[0, 0]


@jax.jit
def _epilogue(part, bias):
    out = pl.pallas_call(
        _epilogue_body,
        out_shape=jax.ShapeDtypeStruct((1, B), jnp.float32),
    )(part, bias.reshape(1, 1))
    return out.reshape(B, 1)


def kernel(user_id, item_id, category_id, shop_id, hist_item_id, target_item_id,
           w_user_id, w_item_id, w_category_id, w_shop_id, w_hist_item_id,
           w_target_item_id, bias):
    ids_all = jnp.stack([
        user_id, item_id, category_id, shop_id, hist_item_id, target_item_id,
    ]).astype(jnp.int32).reshape(NFEAT, B * L)
    w_all = jnp.stack([
        w_user_id, w_item_id, w_category_id, w_shop_id, w_hist_item_id,
        w_target_item_id,
    ])
    part = _sc_partials(ids_all, w_all)
    return _epilogue(part, bias)


# speedup vs baseline: 80.6257x; 1.3702x over previous
import functools

import jax
import jax.numpy as jnp
from jax import lax
from jax.experimental import pallas as pl
from jax.experimental.pallas import tpu as pltpu
from jax.experimental.pallas import tpu_sc as plsc

B = 16384
L = 20
NBUCKETS = 100000
NFEAT = 6

NC = 2
NS = 16
NW = NC * NS

CHUNK = 256
CPF = B // CHUNK
NCHUNKS = NFEAT * CPF
CPW = NCHUNKS // NW
GPC = CHUNK // 16


def _hash16(x):
    h = x.astype(jnp.uint32)
    h = h * jnp.uint32(2654435761)
    h = h ^ (h >> 16)
    h = h * jnp.uint32(2246822519)
    h = h ^ (h >> 13)
    return (h % jnp.uint32(NBUCKETS)).astype(jnp.int32)


def _sc_body(*refs):
    ids_refs = refs[0:NFEAT]
    w_refs = refs[NFEAT:2 * NFEAT]
    part_hbm = refs[2 * NFEAT]
    table_v, ids_v, out_v = refs[2 * NFEAT + 1:]

    wid = lax.axis_index("c") * NS + lax.axis_index("s")
    c_lo = wid * CPW
    c_hi = c_lo + CPW

    def load_table(f):
        for i in range(NFEAT):
            @pl.when(f == i)
            def _load():
                pltpu.sync_copy(w_refs[i], table_v)

    def do_chunk(c, _):
        f = c // CPF
        r0 = (c % CPF) * CHUNK
        for i in range(NFEAT):
            @pl.when(f == i)
            def _load_ids():
                pltpu.sync_copy(ids_refs[i].at[pl.ds(r0 * L, CHUNK * L)], ids_v)

        def group(g, _):
            rows = g * 16 + lax.iota(jnp.int32, 16)
            base = rows * L
            hs = []
            acc = jnp.zeros((16,), jnp.float32)
            for j in range(L):
                idj = plsc.load_gather(ids_v, [base + j])
                h = _hash16(idj)
                wj = plsc.load_gather(table_v, [h])
                if j == 0:
                    acc = wj
                else:
                    m = hs[0] != h
                    for k in range(1, j):
                        m = m & (hs[k] != h)
                    acc = acc + jnp.where(m, wj, 0.0)
                hs.append(h)
            out_v[pl.ds(g * 16, 16)] = acc
            return 0

        lax.fori_loop(0, GPC, group, 0)
        pltpu.sync_copy(out_v, part_hbm.at[f, pl.ds(r0, CHUNK)])
        return 0

    f0 = c_lo // CPF
    f1 = (c_hi - 1) // CPF
    split = jnp.minimum(c_hi, (f0 + 1) * CPF)

    load_table(f0)
    lax.fori_loop(c_lo, split, do_chunk, 0)

    @pl.when(f1 != f0)
    def _second_feature():
        load_table(f1)
        lax.fori_loop(split, c_hi, do_chunk, 0)


@jax.jit
def _sc_partials(*arrays):
    mesh = plsc.VectorSubcoreMesh(core_axis_name="c", subcore_axis_name="s")
    return pl.kernel(
        _sc_body,
        out_type=jax.ShapeDtypeStruct((NFEAT, B), jnp.float32),
        mesh=mesh,
        scratch_types=[
            pltpu.VMEM((NBUCKETS,), jnp.float32),
            pltpu.VMEM((CHUNK * L,), jnp.int32),
            pltpu.VMEM((CHUNK,), jnp.float32),
        ],
        compiler_params=pltpu.CompilerParams(needs_layout_passes=False),
    )(*arrays)


def _epilogue_body(part_ref, bias_ref, out_ref):
    out_ref[:, :] = jnp.sum(part_ref[:, :], axis=0, keepdims=True) + bias_ref[0, 0]


@jax.jit
def _epilogue(part, bias):
    out = pl.pallas_call(
        _epilogue_body,
        out_shape=jax.ShapeDtypeStruct((1, B), jnp.float32),
    )(part, bias.reshape(1, 1))
    return out.reshape(B, 1)


def kernel(user_id, item_id, category_id, shop_id, hist_item_id, target_item_id,
           w_user_id, w_item_id, w_category_id, w_shop_id, w_hist_item_id,
           w_target_item_id, bias):
    ids = [user_id, item_id, category_id, shop_id, hist_item_id, target_item_id]
    ids = [x.astype(jnp.int32).reshape(B * L) for x in ids]
    ws = [w_user_id, w_item_id, w_category_id, w_shop_id, w_hist_item_id,
          w_target_item_id]
    part = _sc_partials(*ids, *ws)
    return _epilogue(part, bias)


# speedup vs baseline: 80.8112x; 1.0023x over previous
import functools

import jax
import jax.numpy as jnp
from jax import lax
from jax.experimental import pallas as pl
from jax.experimental.pallas import tpu as pltpu
from jax.experimental.pallas import tpu_sc as plsc

B = 16384
L = 20
NBUCKETS = 100000
NFEAT = 6

NC = 2
NS = 16
NW = NC * NS

CHUNK = 256
CPF = B // CHUNK
NCHUNKS = NFEAT * CPF
CPW = NCHUNKS // NW
GPC = CHUNK // 16


def _hash16(x):
    h = x.astype(jnp.uint32)
    h = h * jnp.uint32(2654435761)
    h = h ^ (h >> 16)
    h = h * jnp.uint32(2246822519)
    h = h ^ (h >> 13)
    return h % jnp.uint32(NBUCKETS)


def _sc_body(*refs):
    ids_refs = refs[0:NFEAT]
    w_refs = refs[NFEAT:2 * NFEAT]
    part_hbm = refs[2 * NFEAT]
    table_v, ids_v, out_v = refs[2 * NFEAT + 1:]

    wid = lax.axis_index("c") * NS + lax.axis_index("s")
    c_lo = wid * CPW
    c_hi = c_lo + CPW

    def load_table(f):
        for i in range(NFEAT):
            @pl.when(f == i)
            def _load():
                pltpu.sync_copy(w_refs[i], table_v)

    def do_chunk(c, _):
        f = c // CPF
        r0 = (c % CPF) * CHUNK
        for i in range(NFEAT):
            @pl.when(f == i)
            def _load_ids():
                pltpu.sync_copy(ids_refs[i].at[pl.ds(r0 * L, CHUNK * L)], ids_v)

        def group(g, _):
            rows = g * 16 + lax.iota(jnp.int32, 16)
            base = rows * L
            hs = []
            acc = jnp.zeros((16,), jnp.float32)
            for j in range(L):
                idj = plsc.load_gather(ids_v, [base + j])
                h = _hash16(idj)
                wj = plsc.load_gather(table_v, [h.astype(jnp.int32)])
                if j == 0:
                    acc = wj
                else:
                    md = hs[0] ^ h
                    for k in range(1, j):
                        md = jnp.minimum(md, hs[k] ^ h)
                    acc = acc + jnp.where(md != 0, wj, 0.0)
                hs.append(h)
            out_v[pl.ds(g * 16, 16)] = acc
            return 0

        lax.fori_loop(0, GPC, group, 0)
        pltpu.sync_copy(out_v, part_hbm.at[f, pl.ds(r0, CHUNK)])
        return 0

    f0 = c_lo // CPF
    f1 = (c_hi - 1) // CPF
    split = jnp.minimum(c_hi, (f0 + 1) * CPF)

    load_table(f0)
    lax.fori_loop(c_lo, split, do_chunk, 0)

    @pl.when(f1 != f0)
    def _second_feature():
        load_table(f1)
        lax.fori_loop(split, c_hi, do_chunk, 0)


@jax.jit
def _sc_partials(*arrays):
    mesh = plsc.VectorSubcoreMesh(core_axis_name="c", subcore_axis_name="s")
    return pl.kernel(
        _sc_body,
        out_type=jax.ShapeDtypeStruct((NFEAT, B), jnp.float32),
        mesh=mesh,
        scratch_types=[
            pltpu.VMEM((NBUCKETS,), jnp.float32),
            pltpu.VMEM((CHUNK * L,), jnp.int32),
            pltpu.VMEM((CHUNK,), jnp.float32),
        ],
        compiler_params=pltpu.CompilerParams(needs_layout_passes=False),
    )(*arrays)


def _epilogue_body(part_ref, bias_ref, out_ref):
    out_ref[:, :] = jnp.sum(part_ref[:, :], axis=0, keepdims=True) + bias_ref[0, 0]


@jax.jit
def _epilogue(part, bias):
    out = pl.pallas_call(
        _epilogue_body,
        out_shape=jax.ShapeDtypeStruct((1, B), jnp.float32),
    )(part, bias.reshape(1, 1))
    return out.reshape(B, 1)


def kernel(user_id, item_id, category_id, shop_id, hist_item_id, target_item_id,
           w_user_id, w_item_id, w_category_id, w_shop_id, w_hist_item_id,
           w_target_item_id, bias):
    ids = [user_id, item_id, category_id, shop_id, hist_item_id, target_item_id]
    ids = [x.astype(jnp.int32).reshape(B * L) for x in ids]
    ws = [w_user_id, w_item_id, w_category_id, w_shop_id, w_hist_item_id,
          w_target_item_id]
    part = _sc_partials(*ids, *ws)
    return _epilogue(part, bias)
